# baseline (speedup 1.0000x reference)
import functools

import jax
import jax.numpy as jnp
from jax.experimental import pallas as pl
from jax.experimental.pallas import tpu as pltpu

N = 10000
DEG = 32
D = 128
OUT = 128
BN = 200


def _pass1_body(neigh_ref, self_ref, nn_ref, w_self_ref, b_self_ref,
                w_neigh_ref, b_neigh_ref, h_ref, s1_ref, s2_ref):
    neigh = neigh_ref[...]
    agg = jnp.sum(neigh, axis=1)
    nn = nn_ref[...]
    nn = jnp.where(nn == 0.0, 1.0, nn)
    agg = agg / nn
    self_h = jnp.dot(self_ref[...], w_self_ref[...],
                     preferred_element_type=jnp.float32) + b_self_ref[...]
    agg_h = jnp.dot(agg, w_neigh_ref[...],
                    preferred_element_type=jnp.float32) + b_neigh_ref[...]
    h = jnp.concatenate([self_h, agg_h], axis=1)
    h_ref[...] = h
    ps1 = jnp.sum(h, axis=0, keepdims=True)
    ps2 = jnp.sum(h * h, axis=0, keepdims=True)

    @pl.when(pl.program_id(0) == 0)
    def _init():
        s1_ref[...] = ps1
        s2_ref[...] = ps2

    @pl.when(pl.program_id(0) != 0)
    def _acc():
        s1_ref[...] += ps1
        s2_ref[...] += ps2


def _pass2_body(h_ref, s1_ref, s2_ref, gamma_ref, beta_ref, out_ref):
    mean = s1_ref[...] / N
    var = s2_ref[...] / N - mean * mean
    scale = gamma_ref[...] * jax.lax.rsqrt(var + 1e-3)
    shift = beta_ref[...] - mean * scale
    out_ref[...] = jnp.maximum(h_ref[...] * scale + shift, 0.0)


def kernel(self_feats, neigh_feats, self_nneigh, neigh_nneigh,
           W_self, b_self, W_neigh, b_neigh, gamma, beta):
    nn2 = self_nneigh.reshape(N, 1)
    b_self2 = b_self.reshape(1, OUT)
    b_neigh2 = b_neigh.reshape(1, OUT)
    gamma2 = gamma.reshape(1, 2 * OUT)
    beta2 = beta.reshape(1, 2 * OUT)

    grid = N // BN
    h, s1, s2 = pl.pallas_call(
        _pass1_body,
        grid=(grid,),
        in_specs=[
            pl.BlockSpec((BN, DEG, D), lambda i: (i, 0, 0)),
            pl.BlockSpec((BN, D), lambda i: (i, 0)),
            pl.BlockSpec((BN, 1), lambda i: (i, 0)),
            pl.BlockSpec((D, OUT), lambda i: (0, 0)),
            pl.BlockSpec((1, OUT), lambda i: (0, 0)),
            pl.BlockSpec((D, OUT), lambda i: (0, 0)),
            pl.BlockSpec((1, OUT), lambda i: (0, 0)),
        ],
        out_specs=[
            pl.BlockSpec((BN, 2 * OUT), lambda i: (i, 0)),
            pl.BlockSpec((1, 2 * OUT), lambda i: (0, 0)),
            pl.BlockSpec((1, 2 * OUT), lambda i: (0, 0)),
        ],
        out_shape=[
            jax.ShapeDtypeStruct((N, 2 * OUT), jnp.float32),
            jax.ShapeDtypeStruct((1, 2 * OUT), jnp.float32),
            jax.ShapeDtypeStruct((1, 2 * OUT), jnp.float32),
        ],
    )(neigh_feats, self_feats, nn2, W_self, b_self2, W_neigh, b_neigh2)

    BN2 = 1000
    out = pl.pallas_call(
        _pass2_body,
        grid=(N // BN2,),
        in_specs=[
            pl.BlockSpec((BN2, 2 * OUT), lambda i: (i, 0)),
            pl.BlockSpec((1, 2 * OUT), lambda i: (0, 0)),
            pl.BlockSpec((1, 2 * OUT), lambda i: (0, 0)),
            pl.BlockSpec((1, 2 * OUT), lambda i: (0, 0)),
            pl.BlockSpec((1, 2 * OUT), lambda i: (0, 0)),
        ],
        out_specs=pl.BlockSpec((BN2, 2 * OUT), lambda i: (i, 0)),
        out_shape=jax.ShapeDtypeStruct((N, 2 * OUT), jnp.float32),
    )(h, s1, s2, gamma2, beta2)
    return out


# speedup vs baseline: 1.1755x; 1.1755x over previous
import functools

import jax
import jax.numpy as jnp
from jax.experimental import pallas as pl
from jax.experimental.pallas import tpu as pltpu

N = 10000
DEG = 32
D = 128
OUT = 128
BN = 400


def _pass1_body(neigh_ref, self_ref, nn_ref, w_self_ref, b_self_ref,
                w_neigh_ref, b_neigh_ref, h_ref, s1_ref, s2_ref):
    neigh = neigh_ref[...]
    agg = jnp.sum(neigh, axis=1)
    nn = nn_ref[...]
    nn = jnp.where(nn == 0.0, 1.0, nn)
    agg = agg / nn
    self_h = jnp.dot(self_ref[...], w_self_ref[...],
                     preferred_element_type=jnp.float32) + b_self_ref[...]
    agg_h = jnp.dot(agg, w_neigh_ref[...],
                    preferred_element_type=jnp.float32) + b_neigh_ref[...]
    h = jnp.concatenate([self_h, agg_h], axis=1)
    h_ref[...] = h
    ps1 = jnp.sum(h, axis=0, keepdims=True)
    ps2 = jnp.sum(h * h, axis=0, keepdims=True)

    @pl.when(pl.program_id(0) == 0)
    def _init():
        s1_ref[...] = ps1
        s2_ref[...] = ps2

    @pl.when(pl.program_id(0) != 0)
    def _acc():
        s1_ref[...] += ps1
        s2_ref[...] += ps2


def _pass2_body(h_ref, s1_ref, s2_ref, gamma_ref, beta_ref, out_ref):
    mean = s1_ref[...] / N
    var = s2_ref[...] / N - mean * mean
    scale = gamma_ref[...] * jax.lax.rsqrt(var + 1e-3)
    shift = beta_ref[...] - mean * scale
    out_ref[...] = jnp.maximum(h_ref[...] * scale + shift, 0.0)


def kernel(self_feats, neigh_feats, self_nneigh, neigh_nneigh,
           W_self, b_self, W_neigh, b_neigh, gamma, beta):
    nn2 = self_nneigh.reshape(N, 1)
    b_self2 = b_self.reshape(1, OUT)
    b_neigh2 = b_neigh.reshape(1, OUT)
    gamma2 = gamma.reshape(1, 2 * OUT)
    beta2 = beta.reshape(1, 2 * OUT)

    grid = N // BN
    h, s1, s2 = pl.pallas_call(
        _pass1_body,
        grid=(grid,),
        in_specs=[
            pl.BlockSpec((BN, DEG, D), lambda i: (i, 0, 0)),
            pl.BlockSpec((BN, D), lambda i: (i, 0)),
            pl.BlockSpec((BN, 1), lambda i: (i, 0)),
            pl.BlockSpec((D, OUT), lambda i: (0, 0)),
            pl.BlockSpec((1, OUT), lambda i: (0, 0)),
            pl.BlockSpec((D, OUT), lambda i: (0, 0)),
            pl.BlockSpec((1, OUT), lambda i: (0, 0)),
        ],
        out_specs=[
            pl.BlockSpec((BN, 2 * OUT), lambda i: (i, 0)),
            pl.BlockSpec((1, 2 * OUT), lambda i: (0, 0)),
            pl.BlockSpec((1, 2 * OUT), lambda i: (0, 0)),
        ],
        out_shape=[
            jax.ShapeDtypeStruct((N, 2 * OUT), jnp.float32),
            jax.ShapeDtypeStruct((1, 2 * OUT), jnp.float32),
            jax.ShapeDtypeStruct((1, 2 * OUT), jnp.float32),
        ],
    )(neigh_feats, self_feats, nn2, W_self, b_self2, W_neigh, b_neigh2)

    BN2 = 1000
    out = pl.pallas_call(
        _pass2_body,
        grid=(N // BN2,),
        in_specs=[
            pl.BlockSpec((BN2, 2 * OUT), lambda i: (i, 0)),
            pl.BlockSpec((1, 2 * OUT), lambda i: (0, 0)),
            pl.BlockSpec((1, 2 * OUT), lambda i: (0, 0)),
            pl.BlockSpec((1, 2 * OUT), lambda i: (0, 0)),
            pl.BlockSpec((1, 2 * OUT), lambda i: (0, 0)),
        ],
        out_specs=pl.BlockSpec((BN2, 2 * OUT), lambda i: (i, 0)),
        out_shape=jax.ShapeDtypeStruct((N, 2 * OUT), jnp.float32),
    )(h, s1, s2, gamma2, beta2)
    return out
